# Initial kernel scaffold; baseline (speedup 1.0000x reference)
import jax
import jax.numpy as jnp
from jax.experimental import pallas as pl


def kernel(x, edge_attr, params, edge_index):
    raise NotImplementedError("write your pallas kernel here")


# speedup vs baseline: 3.1293x; 3.1293x over previous
import functools

import jax
import jax.numpy as jnp
from jax.experimental import pallas as pl
from jax.experimental.pallas import tpu as pltpu

_N = 10000
_H = 8
_C = 256
_HD = _H * _C

_BN = 512
_NP = 10240
_BE = 1024


def _mm_apack_body(x_ref, w_ref, ap_ref, out_ref, a_ref):
    xb = x_ref[...]
    out_ref[...] = jnp.dot(xb, w_ref[...], preferred_element_type=jnp.float32)
    a_ref[...] = jnp.dot(xb, ap_ref[...], preferred_element_type=jnp.float32)


def _mm_apack(x, w, apack):
    n, k = x.shape
    nb = n // _BN
    return pl.pallas_call(
        _mm_apack_body,
        grid=(nb,),
        in_specs=[
            pl.BlockSpec((_BN, k), lambda i: (i, 0)),
            pl.BlockSpec((k, _HD), lambda i: (0, 0)),
            pl.BlockSpec((k, 16), lambda i: (0, 0)),
        ],
        out_specs=[
            pl.BlockSpec((_BN, _HD), lambda i: (i, 0)),
            pl.BlockSpec((_BN, 16), lambda i: (i, 0)),
        ],
        out_shape=[
            jax.ShapeDtypeStruct((n, _HD), jnp.float32),
            jax.ShapeDtypeStruct((n, 16), jnp.float32),
        ],
    )(x, w, apack)


def _alpha_body(asg_ref, adg_ref, ea_ref, c_ref, alpha_ref, amax_ref):
    i = pl.program_id(0)
    c_col = c_ref[:, 0:1]
    a = asg_ref[...] + adg_ref[...] + c_col * ea_ref[...]
    a = jnp.where(a > 0, a, 0.2 * a)
    alpha_ref[...] = a
    bm = jnp.max(a, axis=1, keepdims=True)
    bm = jnp.broadcast_to(bm, (8, 128))

    @pl.when(i == 0)
    def _():
        amax_ref[...] = bm

    @pl.when(i > 0)
    def _():
        amax_ref[...] = jnp.maximum(amax_ref[...], bm)


def _alpha(asg, adg, ea, c128):
    ep = asg.shape[1]
    nb = ep // _BE
    return pl.pallas_call(
        _alpha_body,
        grid=(nb,),
        in_specs=[
            pl.BlockSpec((8, _BE), lambda i: (0, i)),
            pl.BlockSpec((8, _BE), lambda i: (0, i)),
            pl.BlockSpec((8, _BE), lambda i: (0, i)),
            pl.BlockSpec((8, 128), lambda i: (0, 0)),
        ],
        out_specs=[
            pl.BlockSpec((8, _BE), lambda i: (0, i)),
            pl.BlockSpec((8, 128), lambda i: (0, 0)),
        ],
        out_shape=[
            jax.ShapeDtypeStruct((8, ep), jnp.float32),
            jax.ShapeDtypeStruct((8, 128), jnp.float32),
        ],
        compiler_params=pltpu.CompilerParams(
            dimension_semantics=("arbitrary",)),
    )(asg, adg, ea, c128)


def _scale_body(g_ref, alpha_ref, amax_ref, r_ref, m_ref, ex_ref):
    shift = amax_ref[:, 0:1]
    ex = jnp.exp(alpha_ref[...] - shift)
    ex_ref[...] = ex
    ext = jnp.swapaxes(ex, 0, 1)
    expand = jnp.dot(ext, r_ref[...], preferred_element_type=jnp.float32)
    m_ref[...] = g_ref[...] * expand


def _scale(g, alpha, amax, r):
    ep = g.shape[0]
    nb = ep // _BE
    return pl.pallas_call(
        _scale_body,
        grid=(nb,),
        in_specs=[
            pl.BlockSpec((_BE, _HD), lambda i: (i, 0)),
            pl.BlockSpec((8, _BE), lambda i: (0, i)),
            pl.BlockSpec((8, 128), lambda i: (0, 0)),
            pl.BlockSpec((8, _HD), lambda i: (0, 0)),
        ],
        out_specs=[
            pl.BlockSpec((_BE, _HD), lambda i: (i, 0)),
            pl.BlockSpec((8, _BE), lambda i: (0, i)),
        ],
        out_shape=[
            jax.ShapeDtypeStruct((ep, _HD), jnp.float32),
            jax.ShapeDtypeStruct((8, ep), jnp.float32),
        ],
    )(g, alpha, amax, r)


def _finalize_body(relu, macc_ref, den_ref, bias_ref, r_ref, h_ref, gsum_ref):
    i = pl.program_id(0)
    inv = 1.0 / (den_ref[...] + 1e-16)
    invt = jnp.swapaxes(inv, 0, 1)
    expand = jnp.dot(invt, r_ref[...], preferred_element_type=jnp.float32)
    h = bias_ref[...] + macc_ref[...] * expand
    if relu:
        h = jnp.maximum(h, 0.0)
    h_ref[...] = h
    rows = jax.lax.broadcasted_iota(jnp.int32, (_BN, 1), 0) + i * _BN
    hm = jnp.where(rows < _N, h, 0.0)
    bs = jnp.sum(hm, axis=0, keepdims=True)
    bs = jnp.broadcast_to(bs, (8, _HD))

    @pl.when(i == 0)
    def _():
        gsum_ref[...] = bs

    @pl.when(i > 0)
    def _():
        gsum_ref[...] = gsum_ref[...] + bs


def _finalize(macc, den, bias, r, relu):
    n = macc.shape[0]
    nb = n // _BN
    return pl.pallas_call(
        functools.partial(_finalize_body, relu),
        grid=(nb,),
        in_specs=[
            pl.BlockSpec((_BN, _HD), lambda i: (i, 0)),
            pl.BlockSpec((8, _BN), lambda i: (0, i)),
            pl.BlockSpec((1, _HD), lambda i: (0, 0)),
            pl.BlockSpec((8, _HD), lambda i: (0, 0)),
        ],
        out_specs=[
            pl.BlockSpec((_BN, _HD), lambda i: (i, 0)),
            pl.BlockSpec((8, _HD), lambda i: (0, 0)),
        ],
        out_shape=[
            jax.ShapeDtypeStruct((n, _HD), jnp.float32),
            jax.ShapeDtypeStruct((8, _HD), jnp.float32),
        ],
        compiler_params=pltpu.CompilerParams(
            dimension_semantics=("arbitrary",)),
    )(macc, den, bias, r)


def _heads_body(gs_ref, wp1_ref, bp1_ref, gp1_ref, blp1_ref, wp2_ref, bp2_ref,
                wv1_ref, bv1_ref, gv1_ref, blv1_ref, wv2_ref, bv2_ref,
                pol_ref, val_ref):
    g = gs_ref[0:1, :] * (1.0 / _N)

    def ln(x, gamma, beta):
        m = jnp.mean(x, axis=1, keepdims=True)
        v = jnp.mean((x - m) ** 2, axis=1, keepdims=True)
        return (x - m) * jax.lax.rsqrt(v + 1e-5) * gamma + beta

    p1 = jnp.dot(g, wp1_ref[...], preferred_element_type=jnp.float32)
    p1 = jnp.maximum(ln(p1 + bp1_ref[...], gp1_ref[...], blp1_ref[...]), 0.0)
    logits = jnp.dot(p1, wp2_ref[...], preferred_element_type=jnp.float32)
    logits = logits + bp2_ref[...]
    lm = jnp.max(logits, axis=1, keepdims=True)
    el = jnp.exp(logits - lm)
    pol = el / jnp.sum(el, axis=1, keepdims=True)
    pol_ref[...] = jnp.broadcast_to(pol, (8, 128))

    v1 = jnp.dot(g, wv1_ref[...], preferred_element_type=jnp.float32)
    v1 = jnp.maximum(ln(v1 + bv1_ref[...], gv1_ref[...], blv1_ref[...]), 0.0)
    v = jnp.dot(v1, wv2_ref[...], preferred_element_type=jnp.float32)
    v = jnp.tanh(v + bv2_ref[...])
    val_ref[...] = jnp.broadcast_to(v, (8, 128))


def _heads(gsum, p):
    full = lambda s: pl.BlockSpec(s, lambda: tuple(0 for _ in s))
    wv2 = p['Wv2'].reshape(512, 1)
    args = [gsum,
            p['Wp1'], p['bp1'].reshape(1, 512), p['gp1'].reshape(1, 512),
            p['bp_ln1'].reshape(1, 512), p['Wp2'], p['bp2'].reshape(1, 128),
            p['Wv1'], p['bv1'].reshape(1, 512), p['gv1'].reshape(1, 512),
            p['bv_ln1'].reshape(1, 512), wv2, p['bv2'].reshape(1, 1)]
    return pl.pallas_call(
        _heads_body,
        in_specs=[full(a.shape) for a in args],
        out_specs=[full((8, 128)), full((8, 128))],
        out_shape=[jax.ShapeDtypeStruct((8, 128), jnp.float32),
                   jax.ShapeDtypeStruct((8, 128), jnp.float32)],
    )(*args)


def _pad_to(x, m, axis, value):
    n = x.shape[axis]
    r = (-n) % m
    if r == 0:
        return x
    widths = [(0, 0)] * x.ndim
    widths[axis] = (0, r)
    return jnp.pad(x, widths, constant_values=value)


def _gat_layer(xs, apack, src, dst, ea, c128, r, bias, relu):
    at_s = jnp.take(apack, src, axis=0)
    at_d = jnp.take(apack, dst.clip(0, _N - 1), axis=0)
    asg = at_s[:, 0:8].T
    adg = at_d[:, 8:16].T
    ea8 = jnp.broadcast_to(ea.reshape(1, -1), (8, ea.shape[0]))
    alpha, amax = _alpha(asg, adg, ea8, c128)
    g = jnp.take(xs, src, axis=0)
    m, ex = _scale(g, alpha, amax, r)
    den = jax.ops.segment_sum(ex.T, dst, _N)
    macc = jax.ops.segment_sum(m, dst, _N)
    maccp = _pad_to(macc, _BN, 0, 0.0)
    denp = _pad_to(den.T, _BN, 1, 0.0)
    return _finalize(maccp, denp, bias.reshape(1, _HD), r, relu)


def kernel(x, edge_attr, params, edge_index):
    p1, p2 = params['conv1'], params['conv2']
    src, dst = edge_index[0], edge_index[1]
    e = src.shape[0]
    ea = edge_attr[:, 0]

    def apack_mat(pp):
        return pp['W'] @ jnp.concatenate([
            jax.scipy.linalg.block_diag(
                *[pp['att_src'][h].reshape(_C, 1) for h in range(_H)]),
            jax.scipy.linalg.block_diag(
                *[pp['att_dst'][h].reshape(_C, 1) for h in range(_H)]),
        ], axis=1)

    def c_of(pp):
        c = (pp['W_edge'].reshape(_H, _C) * pp['att_edge']).sum(-1)
        return jnp.broadcast_to(c.reshape(8, 1), (8, 128))

    r = jnp.repeat(jnp.eye(_H, dtype=jnp.float32), _C, axis=1)

    xp = _pad_to(x, _BN, 0, 0.0)
    xs1, ap1 = _mm_apack(xp, p1['W'], apack_mat(p1))
    srcp = _pad_to(src, _BE, 0, 0)
    dstp = _pad_to(dst, _BE, 0, _N)
    eap = _pad_to(ea, _BE, 0, 0.0)
    h1, _ = _gat_layer(xs1, ap1, srcp, dstp, eap, c_of(p1), r,
                       p1['bias'], True)

    ones = jnp.ones((e,), jnp.float32)
    cnt = jax.ops.segment_sum(ones, dst, _N)
    sm = jax.ops.segment_sum(ea, dst, _N)
    loop_attr = sm / jnp.maximum(cnt, 1.0)
    loop = jnp.arange(_N, dtype=src.dtype)
    src2 = _pad_to(jnp.concatenate([src, loop]), _BE, 0, 0)
    dst2 = _pad_to(jnp.concatenate([dst, loop]), _BE, 0, _N)
    ea2 = _pad_to(jnp.concatenate([ea, loop_attr]), _BE, 0, 0.0)

    xs2, ap2 = _mm_apack(h1, p2['W'], apack_mat(p2))
    h2, gsum = _gat_layer(xs2, ap2, src2, dst2, ea2, c_of(p2), r,
                          p2['bias'], False)

    pol, val = _heads(gsum, params)
    return pol[0], val[0, 0:1]
